# Initial kernel scaffold; baseline (speedup 1.0000x reference)
import jax
import jax.numpy as jnp
from jax.experimental import pallas as pl


def kernel(x, edge_index, edge_attr, Wv1, Ww1, Wu1, Wa1, Wv2, Ww2, Wu2, Wa2):
    raise NotImplementedError("write your pallas kernel here")


# speedup vs baseline: 3.5742x; 3.5742x over previous
import functools

import jax
import jax.numpy as jnp
from jax import lax
from jax.experimental import pallas as pl
from jax.experimental.pallas import tpu as pltpu
from jax.experimental.pallas import tpu_sc as plsc

F32 = jnp.float32

NB = 1000
LANES = 16
NUM_CORES = 2
NUM_SUBCORES = 16
CHUNK = 80
SPLIT = 8960


def _prep_body(h_ref, ww_ref, wu_ref, wsd_ref, z_ref, zi_ref, sd_ref):
    h = h_ref[...]
    dn = (((1,), (1,)), ((), ()))
    z = lax.dot_general(h, ww_ref[...], dn, preferred_element_type=F32)
    z_ref[...] = z
    zi_ref[...] = lax.dot_general(h, wu_ref[...], dn, preferred_element_type=F32)
    sd_ref[...] = jnp.dot(z, wsd_ref[...], preferred_element_type=F32)


@functools.lru_cache(maxsize=None)
def _prep_call(n, hdim, indim):
    return pl.pallas_call(
        _prep_body,
        grid=(n // NB,),
        in_specs=[
            pl.BlockSpec((NB, indim), lambda i: (i, 0)),
            pl.BlockSpec((hdim, indim), lambda i: (0, 0)),
            pl.BlockSpec((hdim, indim), lambda i: (0, 0)),
            pl.BlockSpec((hdim, 128), lambda i: (0, 0)),
        ],
        out_specs=[
            pl.BlockSpec((NB, hdim), lambda i: (i, 0)),
            pl.BlockSpec((NB, hdim), lambda i: (i, 0)),
            pl.BlockSpec((NB, 128), lambda i: (i, 0)),
        ],
        out_shape=[
            jax.ShapeDtypeStruct((n, hdim), F32),
            jax.ShapeDtypeStruct((n, hdim), F32),
            jax.ShapeDtypeStruct((n, 128), F32),
        ],
    )


def _fin_body(zi_ref, az_ref, den_ref, h_ref):
    w = jnp.concatenate([az_ref[0], az_ref[1]], axis=1)
    den = den_ref[...]
    den = jnp.where(den == 0.0, 1.0, den)
    h_ref[...] = jnp.maximum(zi_ref[...] + w / den, 0.0)


@functools.lru_cache(maxsize=None)
def _fin_call(n, hdim):
    return pl.pallas_call(
        _fin_body,
        grid=(n // NB,),
        in_specs=[
            pl.BlockSpec((NB, hdim), lambda i: (i, 0)),
            pl.BlockSpec((NUM_CORES, NB, 128), lambda i: (0, i, 0)),
            pl.BlockSpec((NB, 1), lambda i: (i, 0)),
        ],
        out_specs=pl.BlockSpec((NB, hdim), lambda i: (i, 0)),
        out_shape=jax.ShapeDtypeStruct((n, hdim), F32),
    )


@functools.lru_cache(maxsize=None)
def _edge_call(n, e):
    tail = n - SPLIT
    nd_rows = -(-n // 128) + 1
    e_per_tile = e // NUM_SUBCORES
    n_chunks = e_per_tile // CHUNK
    half = n_chunks // 2
    g16 = CHUNK // LANES
    t_rows = SPLIT // NUM_SUBCORES

    mesh = plsc.VectorSubcoreMesh(
        core_axis_name="c", subcore_axis_name="s",
        num_cores=NUM_CORES, num_subcores=NUM_SUBCORES)

    def body(z_hbm, s_hbm, d_hbm, src_hbm, dst_hbm, ea_hbm, cm_hbm,
             outz_hbm, outd_hbm,
             s_v, d_v, cm_v, srcb, dstb, gidx, rowzb, dstdb, eab, pv,
             rows, rows_den, zbuf, accz, accd):
        core = lax.axis_index("c")
        tile = lax.axis_index("s")
        zv = jnp.zeros((LANES,), F32)

        @pl.loop(0, 128)
        def _zb(r):
            for q in range(8):
                zbuf[r, pl.ds(q * LANES, LANES)] = zv

        @pl.loop(0, CHUNK)
        def _zd(r):
            for q in range(8):
                rows_den[r, pl.ds(q * LANES, LANES)] = zv

        def zero_acc():
            for j in range(4):
                pltpu.sync_copy(
                    zbuf, accz.at[pl.ds(tile * t_rows + j * 128, 128)])
            pltpu.sync_copy(zbuf.at[pl.ds(0, 48)],
                            accz.at[pl.ds(tile * t_rows + 512, 48)])

            @pl.when(tile == 0)
            def _tr():
                pltpu.sync_copy(zbuf.at[pl.ds(0, 8)],
                                accz.at[pl.ds(SPLIT, 8)])

        zero_acc()

        @pl.when(tile == 0)
        def _initd():
            pltpu.sync_copy(rows_den, accd)

        pltpu.sync_copy(s_hbm, s_v)
        pltpu.sync_copy(d_hbm, d_v)
        pltpu.sync_copy(cm_hbm, cm_v)
        c1v = cm_v[pl.ds(0, LANES)]
        mv = cm_v[pl.ds(LANES, LANES)]
        lane_iota = lax.iota(jnp.int32, LANES)

        plsc.subcore_barrier()

        tile_base = tile * e_per_tile

        for phase in range(2):
            @pl.loop(0, n_chunks)
            def _chunk(i):
                duty = lax.select(core == 0, i < half, i >= half)
                base = tile_base + i * CHUNK
                pltpu.sync_copy(src_hbm.at[pl.ds(base, CHUNK)], srcb)
                pltpu.sync_copy(dst_hbm.at[pl.ds(base, CHUNK)], dstb)
                pltpu.sync_copy(ea_hbm.at[pl.ds(base, CHUNK)], eab)

                @pl.loop(0, g16)
                def _pcalc(g):
                    o = g * LANES
                    src16 = srcb[pl.ds(o, LANES)]
                    dst16 = dstb[pl.ds(o, LANES)]
                    ea16 = eab[pl.ds(o, LANES)]
                    gidx[pl.ds(o, LANES)] = src16 * 2 + core
                    if phase == 0:
                        rowzb[pl.ds(o, LANES)] = jnp.minimum(dst16, SPLIT)
                        dstdb[pl.ds(o, LANES)] = (
                            lax.shift_right_logical(dst16, 7))
                    else:
                        rowzb[pl.ds(o, LANES)] = jnp.where(
                            dst16 >= SPLIT, dst16 - SPLIT, tail)
                    sg = plsc.load_gather(s_v, [src16])
                    dg = plsc.load_gather(d_v, [dst16])
                    a = sg + dg + c1v * ea16
                    ev = jnp.maximum(a, a * 0.01)
                    pv[pl.ds(o, LANES)] = jnp.exp(ev - mv)

                pltpu.sync_copy(z_hbm.at[gidx], rows)

                @pl.loop(0, g16)
                def _scale(g):
                    o = g * LANES
                    pv16 = pv[pl.ds(o, LANES)]
                    for j in range(LANES):
                        pb = jnp.full((LANES,), pv16[j])
                        for q in range(8):
                            sl = pl.ds(q * LANES, LANES)
                            rows[o + j, sl] = rows[o + j, sl] * pb

                pltpu.sync_copy(rows, accz.at[rowzb], add=True)

                if phase == 0:
                    @pl.when(duty)
                    def _den():
                        @pl.loop(0, g16)
                        def _dw(g):
                            o = g * LANES
                            dv16 = dstb[pl.ds(o, LANES)]
                            pv16 = pv[pl.ds(o, LANES)]
                            for j in range(LANES):
                                off = dv16[j] & 0x70
                                sel = lane_iota == (dv16[j] & 15)
                                rows_den[o + j, pl.ds(off, LANES)] = jnp.where(
                                    sel, jnp.full((LANES,), pv16[j]), zv)

                        pltpu.sync_copy(rows_den, accd.at[dstdb], add=True)

                        @pl.loop(0, g16)
                        def _dz(g):
                            o = g * LANES
                            dv16 = dstb[pl.ds(o, LANES)]
                            for j in range(LANES):
                                off = dv16[j] & 0x70
                                rows_den[o + j, pl.ds(off, LANES)] = zv

            plsc.subcore_barrier()

            if phase == 0:
                pltpu.sync_copy(accz.at[pl.ds(tile * t_rows, t_rows)],
                                outz_hbm.at[core, pl.ds(tile * t_rows, t_rows)])

                @pl.when(tile == 0)
                def _outd():
                    pltpu.sync_copy(accd, outd_hbm.at[core])

                plsc.subcore_barrier()
                zero_acc()
                plsc.subcore_barrier()
            else:
                @pl.when(tile < 13)
                def _fl():
                    pltpu.sync_copy(
                        accz.at[pl.ds(tile * CHUNK, CHUNK)],
                        outz_hbm.at[core, pl.ds(SPLIT + tile * CHUNK, CHUNK)])

    npad = SPLIT + 16 * CHUNK
    return pl.kernel(
        body,
        out_type=[
            jax.ShapeDtypeStruct((NUM_CORES, npad, 128), F32),
            jax.ShapeDtypeStruct((NUM_CORES, nd_rows, 128), F32),
        ],
        mesh=mesh,
        compiler_params=pltpu.CompilerParams(needs_layout_passes=False),
        scratch_types=[
            pltpu.VMEM((n,), F32),
            pltpu.VMEM((n,), F32),
            pltpu.VMEM((2 * LANES,), F32),
            pltpu.VMEM((CHUNK,), jnp.int32),
            pltpu.VMEM((CHUNK,), jnp.int32),
            pltpu.VMEM((CHUNK,), jnp.int32),
            pltpu.VMEM((CHUNK,), jnp.int32),
            pltpu.VMEM((CHUNK,), jnp.int32),
            pltpu.VMEM((CHUNK,), F32),
            pltpu.VMEM((CHUNK,), F32),
            pltpu.VMEM((CHUNK, 128), F32),
            pltpu.VMEM((CHUNK, 128), F32),
            pltpu.VMEM((128, 128), F32),
            pltpu.VMEM_SHARED((SPLIT + 8, 128), F32),
            pltpu.VMEM_SHARED((80, 128), F32),
        ],
    )


def _layer(h, ea, src, dst, Wv, Ww, Wu, Wa):
    n, indim = h.shape
    hdim = Ww.shape[0]
    e = src.shape[0]
    wa_s = Wa[0, :hdim]
    wa_d = Wa[0, hdim:2 * hdim]
    c1 = Wv[0, 0] * Wa[0, 2 * hdim]
    wsd = jnp.zeros((hdim, 128), F32).at[:, 0].set(wa_s).at[:, 1].set(wa_d)

    z, zi, sd = _prep_call(n, hdim, indim)(h, Ww, Wu, wsd)
    s = sd[:, 0]
    d = sd[:, 1]
    m = jnp.maximum(jnp.max(s) + jnp.max(d), 0.0)
    cm = jnp.concatenate([jnp.full((LANES,), c1, F32),
                          jnp.full((LANES,), m, F32)])

    z2 = z.reshape(2 * n, 128)
    outz, outd = _edge_call(n, e)(z2, s, d, src, dst, ea, cm)
    nd_rows = -(-n // 128) + 1
    den = (outd[0] + outd[1]).reshape(nd_rows * 128)[:n, None]
    return _fin_call(n, hdim)(zi, outz, den)


def kernel(x, edge_index, edge_attr, Wv1, Ww1, Wu1, Wa1, Wv2, Ww2, Wu2, Wa2):
    src = edge_index[0]
    dst = edge_index[1]
    ea = edge_attr[:, 0]
    h = _layer(x, ea, src, dst, Wv1, Ww1, Wu1, Wa1)
    return _layer(h, ea, src, dst, Wv2, Ww2, Wu2, Wa2)


# speedup vs baseline: 8.7583x; 2.4505x over previous
import functools

import jax
import jax.numpy as jnp
from jax import lax
from jax.experimental import pallas as pl
from jax.experimental.pallas import tpu as pltpu
from jax.experimental.pallas import tpu_sc as plsc

F32 = jnp.float32

NB = 1000
LANES = 16
NUM_CORES = 2
NUM_SUBCORES = 16
CHUNK = 80
SPLIT = 7552


def _prep_body(h_ref, ww_ref, wu_ref, wsd_ref, z_ref, zi_ref, sd_ref):
    h = h_ref[...]
    dn = (((1,), (1,)), ((), ()))
    z = lax.dot_general(h, ww_ref[...], dn, preferred_element_type=F32)
    z_ref[...] = z
    zi_ref[...] = lax.dot_general(h, wu_ref[...], dn, preferred_element_type=F32)
    sd_ref[...] = jnp.dot(z, wsd_ref[...], preferred_element_type=F32)


def _fprep_body(zi_ref, az_ref, den_ref, ww_ref, wu_ref, wsd_ref,
                z_ref, zi2_ref, sd_ref):
    w = jnp.concatenate([az_ref[0], az_ref[1]], axis=1)
    den = den_ref[...]
    den = jnp.where(den == 0.0, 1.0, den)
    h = jnp.maximum(zi_ref[...] + w / den, 0.0)
    dn = (((1,), (1,)), ((), ()))
    z = lax.dot_general(h, ww_ref[...], dn, preferred_element_type=F32)
    z_ref[...] = z
    zi2_ref[...] = lax.dot_general(h, wu_ref[...], dn,
                                   preferred_element_type=F32)
    sd_ref[...] = jnp.dot(z, wsd_ref[...], preferred_element_type=F32)


@functools.lru_cache(maxsize=None)
def _fprep_call(n, hdim):
    return pl.pallas_call(
        _fprep_body,
        grid=(n // NB,),
        in_specs=[
            pl.BlockSpec((NB, hdim), lambda i: (i, 0)),
            pl.BlockSpec((NUM_CORES, NB, 128), lambda i: (0, i, 0)),
            pl.BlockSpec((NB, 1), lambda i: (i, 0)),
            pl.BlockSpec((hdim, hdim), lambda i: (0, 0)),
            pl.BlockSpec((hdim, hdim), lambda i: (0, 0)),
            pl.BlockSpec((hdim, 128), lambda i: (0, 0)),
        ],
        out_specs=[
            pl.BlockSpec((NB, hdim), lambda i: (i, 0)),
            pl.BlockSpec((NB, hdim), lambda i: (i, 0)),
            pl.BlockSpec((NB, 128), lambda i: (i, 0)),
        ],
        out_shape=[
            jax.ShapeDtypeStruct((n, hdim), F32),
            jax.ShapeDtypeStruct((n, hdim), F32),
            jax.ShapeDtypeStruct((n, 128), F32),
        ],
    )


@functools.lru_cache(maxsize=None)
def _prep_call(n, hdim, indim):
    return pl.pallas_call(
        _prep_body,
        grid=(n // NB,),
        in_specs=[
            pl.BlockSpec((NB, indim), lambda i: (i, 0)),
            pl.BlockSpec((hdim, indim), lambda i: (0, 0)),
            pl.BlockSpec((hdim, indim), lambda i: (0, 0)),
            pl.BlockSpec((hdim, 128), lambda i: (0, 0)),
        ],
        out_specs=[
            pl.BlockSpec((NB, hdim), lambda i: (i, 0)),
            pl.BlockSpec((NB, hdim), lambda i: (i, 0)),
            pl.BlockSpec((NB, 128), lambda i: (i, 0)),
        ],
        out_shape=[
            jax.ShapeDtypeStruct((n, hdim), F32),
            jax.ShapeDtypeStruct((n, hdim), F32),
            jax.ShapeDtypeStruct((n, 128), F32),
        ],
    )


def _fin_body(zi_ref, az_ref, den_ref, h_ref):
    w = jnp.concatenate([az_ref[0], az_ref[1]], axis=1)
    den = den_ref[...]
    den = jnp.where(den == 0.0, 1.0, den)
    h_ref[...] = jnp.maximum(zi_ref[...] + w / den, 0.0)


@functools.lru_cache(maxsize=None)
def _fin_call(n, hdim):
    return pl.pallas_call(
        _fin_body,
        grid=(n // NB,),
        in_specs=[
            pl.BlockSpec((NB, hdim), lambda i: (i, 0)),
            pl.BlockSpec((NUM_CORES, NB, 128), lambda i: (0, i, 0)),
            pl.BlockSpec((NB, 1), lambda i: (i, 0)),
        ],
        out_specs=pl.BlockSpec((NB, hdim), lambda i: (i, 0)),
        out_shape=jax.ShapeDtypeStruct((n, hdim), F32),
    )


@functools.lru_cache(maxsize=None)
def _edge_call(n, e):
    tail = n - SPLIT
    nd_rows = -(-n // 128) + 1
    e_per_tile = e // NUM_SUBCORES
    n_chunks = e_per_tile // CHUNK
    half = n_chunks // 2
    g16 = CHUNK // LANES
    t_rows = SPLIT // NUM_SUBCORES

    mesh = plsc.VectorSubcoreMesh(
        core_axis_name="c", subcore_axis_name="s",
        num_cores=NUM_CORES, num_subcores=NUM_SUBCORES)

    def body(z_hbm, s_hbm, d_hbm, src_hbm, dst_hbm, ea_hbm, cm_hbm,
             outz_hbm, outd_hbm,
             s_v, d_v, cm_v,
             srcb0, dstb0, eab0, srcb1, dstb1, eab1,
             gidx0, rowzb0, dstdb0, pv0, gidx1, rowzb1, dstdb1, pv1,
             rows0, rows1, rows_den, zbuf, accz, accd,
             isem0, isem1, gsem0, gsem1, ssem0, ssem1):
        core = lax.axis_index("c")
        tile = lax.axis_index("s")
        zv = jnp.zeros((LANES,), F32)
        inb = ((srcb0, dstb0, eab0, isem0), (srcb1, dstb1, eab1, isem1))
        cbb = ((gidx0, rowzb0, dstdb0, pv0), (gidx1, rowzb1, dstdb1, pv1))
        rwb = ((rows0, gsem0, ssem0), (rows1, gsem1, ssem1))

        @pl.loop(0, 128)
        def _zb(r):
            for q in range(8):
                zbuf[r, pl.ds(q * LANES, LANES)] = zv

        @pl.loop(0, CHUNK)
        def _zd(r):
            for q in range(8):
                rows_den[r, pl.ds(q * LANES, LANES)] = zv

        def zero_acc():
            for j in range(3):
                pltpu.sync_copy(
                    zbuf, accz.at[pl.ds(tile * t_rows + j * 128, 128)])
            pltpu.sync_copy(zbuf.at[pl.ds(0, 88)],
                            accz.at[pl.ds(tile * t_rows + 384, 88)])

            @pl.when(tile == 0)
            def _tr():
                pltpu.sync_copy(zbuf.at[pl.ds(0, 64)],
                                accz.at[pl.ds(SPLIT, 64)])

        zero_acc()

        @pl.when(tile == 0)
        def _initd():
            pltpu.sync_copy(rows_den, accd)

        pltpu.sync_copy(s_hbm, s_v)
        pltpu.sync_copy(d_hbm, d_v)
        pltpu.sync_copy(cm_hbm, cm_v)
        c1v = cm_v[pl.ds(0, LANES)]
        mv = cm_v[pl.ds(LANES, LANES)]
        lane_iota = lax.iota(jnp.int32, LANES)

        plsc.subcore_barrier()

        tile_base = tile * e_per_tile

        def issue_in(i, b):
            base = tile_base + i * CHUNK
            srcb, dstb, eab, sem = inb[b]
            pltpu.async_copy(src_hbm.at[pl.ds(base, CHUNK)], srcb, sem)
            pltpu.async_copy(dst_hbm.at[pl.ds(base, CHUNK)], dstb, sem)
            pltpu.async_copy(ea_hbm.at[pl.ds(base, CHUNK)], eab, sem)

        def wait_in(b):
            srcb, dstb, eab, sem = inb[b]
            pltpu.make_async_copy(
                src_hbm.at[pl.ds(0, CHUNK)], srcb, sem).wait()
            pltpu.make_async_copy(
                dst_hbm.at[pl.ds(0, CHUNK)], dstb, sem).wait()
            pltpu.make_async_copy(
                ea_hbm.at[pl.ds(0, CHUNK)], eab, sem).wait()

        def compute(b, phase):
            srcb, dstb, eab, _ = inb[b]
            gidx, rowzb, dstdb, pv = cbb[b]

            @pl.loop(0, g16)
            def _pcalc(g):
                o = g * LANES
                src16 = srcb[pl.ds(o, LANES)]
                dst16 = dstb[pl.ds(o, LANES)]
                ea16 = eab[pl.ds(o, LANES)]
                gidx[pl.ds(o, LANES)] = src16 * 2 + core
                if phase == 0:
                    rowzb[pl.ds(o, LANES)] = jnp.where(
                        dst16 < SPLIT, dst16, SPLIT + (dst16 & 63))
                    dstdb[pl.ds(o, LANES)] = (
                        lax.shift_right_logical(dst16, 7))
                else:
                    rowzb[pl.ds(o, LANES)] = jnp.where(
                        dst16 >= SPLIT, dst16 - SPLIT,
                        tail + (dst16 & 63))
                sg = plsc.load_gather(s_v, [src16])
                dg = plsc.load_gather(d_v, [dst16])
                a = sg + dg + c1v * ea16
                ev = jnp.maximum(a, a * 0.01)
                pv[pl.ds(o, LANES)] = jnp.exp(ev - mv)

        def issue_gather(b):
            pltpu.async_copy(z_hbm.at[cbb[b][0]], rwb[b][0], rwb[b][1])

        def wait_gather(b):
            pltpu.make_async_copy(
                z_hbm.at[cbb[b][0]], rwb[b][0], rwb[b][1]).wait()

        def issue_scat(b):
            pltpu.async_copy(rwb[b][0], accz.at[cbb[b][1]], rwb[b][2],
                             add=True)

        def wait_scat(b):
            pltpu.make_async_copy(
                rwb[b][0], accz.at[cbb[b][1]], rwb[b][2]).wait()

        def den_work(i, b):
            duty = lax.select(core == 0, i < half, i >= half)
            dstb = inb[b][1]
            dstdb = cbb[b][2]
            pv = cbb[b][3]

            @pl.when(duty)
            def _den():
                @pl.loop(0, g16)
                def _dw(g):
                    o = g * LANES
                    dv16 = dstb[pl.ds(o, LANES)]
                    pv16 = pv[pl.ds(o, LANES)]
                    for j in range(LANES):
                        off = dv16[j] & 0x70
                        sel = lane_iota == (dv16[j] & 15)
                        rows_den[o + j, pl.ds(off, LANES)] = jnp.where(
                            sel, jnp.full((LANES,), pv16[j]), zv)

                pltpu.sync_copy(rows_den, accd.at[dstdb], add=True)

                @pl.loop(0, g16)
                def _dz(g):
                    o = g * LANES
                    dv16 = dstb[pl.ds(o, LANES)]
                    for j in range(LANES):
                        off = dv16[j] & 0x70
                        rows_den[o + j, pl.ds(off, LANES)] = zv

        def scale(b):
            pv = cbb[b][3]
            rows = rwb[b][0]

            @pl.loop(0, g16)
            def _scale(g):
                o = g * LANES
                pv16 = pv[pl.ds(o, LANES)]
                for j in range(LANES):
                    pb = jnp.full((LANES,), pv16[j])
                    for q in range(8):
                        sl = pl.ds(q * LANES, LANES)
                        rows[o + j, sl] = rows[o + j, sl] * pb

        def slot(i, b, phase, skip_scat_wait=False, has_prev=True):
            wait_in(b)
            if not skip_scat_wait:
                wait_scat(b)
            compute(b, phase)
            issue_gather(b)
            if phase == 0:
                den_work(i, b)
            if isinstance(i, int):
                if i + 2 < n_chunks:
                    issue_in(i + 2, b)
            else:
                @pl.when(i + 2 < n_chunks)
                def _pf():
                    issue_in(i + 2, b)
            if has_prev:
                a = 1 - b
                wait_gather(a)
                scale(a)
                issue_scat(a)

        for phase in range(2):
            issue_in(0, 0)
            issue_in(1, 1)
            slot(0, 0, phase, skip_scat_wait=True, has_prev=False)
            slot(1, 1, phase, skip_scat_wait=True)

            @pl.loop(1, (n_chunks - 1) // 2)
            def _pair(k):
                slot(2 * k, 0, phase)
                slot(2 * k + 1, 1, phase)

            slot(n_chunks - 1, 0, phase)
            wait_gather(0)
            scale(0)
            issue_scat(0)
            wait_scat(1)
            wait_scat(0)

            plsc.subcore_barrier()

            if phase == 0:
                pltpu.sync_copy(accz.at[pl.ds(tile * t_rows, t_rows)],
                                outz_hbm.at[core, pl.ds(tile * t_rows, t_rows)])

                @pl.when(tile == 0)
                def _outd():
                    pltpu.sync_copy(accd, outd_hbm.at[core])

                plsc.subcore_barrier()
                zero_acc()
                plsc.subcore_barrier()
            else:
                @pl.when(tile < 15)
                def _fl():
                    pltpu.sync_copy(
                        accz.at[pl.ds(tile * 160, 160)],
                        outz_hbm.at[core, pl.ds(SPLIT + tile * 160, 160)])

                @pl.when(tile == 15)
                def _fl2():
                    pltpu.sync_copy(
                        accz.at[pl.ds(2400, 48)],
                        outz_hbm.at[core, pl.ds(SPLIT + 2400, 48)])

    npad = n
    ibuf = [
        pltpu.VMEM((CHUNK,), jnp.int32),
        pltpu.VMEM((CHUNK,), jnp.int32),
        pltpu.VMEM((CHUNK,), F32),
    ]
    cbuf = [
        pltpu.VMEM((CHUNK,), jnp.int32),
        pltpu.VMEM((CHUNK,), jnp.int32),
        pltpu.VMEM((CHUNK,), jnp.int32),
        pltpu.VMEM((CHUNK,), F32),
    ]
    return pl.kernel(
        body,
        out_type=[
            jax.ShapeDtypeStruct((NUM_CORES, npad, 128), F32),
            jax.ShapeDtypeStruct((NUM_CORES, nd_rows, 128), F32),
        ],
        mesh=mesh,
        compiler_params=pltpu.CompilerParams(needs_layout_passes=False),
        scratch_types=[
            pltpu.VMEM((n,), F32),
            pltpu.VMEM((n,), F32),
            pltpu.VMEM((2 * LANES,), F32),
            *ibuf, *ibuf, *cbuf, *cbuf,
            pltpu.VMEM((CHUNK, 128), F32),
            pltpu.VMEM((CHUNK, 128), F32),
            pltpu.VMEM((CHUNK, 128), F32),
            pltpu.VMEM((128, 128), F32),
            pltpu.VMEM_SHARED((SPLIT + 64, 128), F32),
            pltpu.VMEM_SHARED((80, 128), F32),
            pltpu.SemaphoreType.DMA,
            pltpu.SemaphoreType.DMA,
            pltpu.SemaphoreType.DMA,
            pltpu.SemaphoreType.DMA,
            pltpu.SemaphoreType.DMA,
            pltpu.SemaphoreType.DMA,
        ],
    )


def _wsd(Wa, hdim):
    wa_s = Wa[0, :hdim]
    wa_d = Wa[0, hdim:2 * hdim]
    return jnp.zeros((hdim, 128), F32).at[:, 0].set(wa_s).at[:, 1].set(wa_d)


def _edge_phase(z, s, d, src, dst, ea, c1, n, e):
    m = jnp.maximum(jnp.max(s) + jnp.max(d), 0.0)
    cm = jnp.concatenate([jnp.full((LANES,), c1, F32),
                          jnp.full((LANES,), m, F32)])
    z2 = z.reshape(2 * n, 128)
    outz, outd = _edge_call(n, e)(z2, s, d, src, dst, ea, cm)
    nd_rows = -(-n // 128) + 1
    den = (outd[0] + outd[1]).reshape(nd_rows * 128)[:n, None]
    return outz, den


def kernel(x, edge_index, edge_attr, Wv1, Ww1, Wu1, Wa1, Wv2, Ww2, Wu2, Wa2):
    src = edge_index[0]
    dst = edge_index[1]
    ea = edge_attr[:, 0]
    n, indim = x.shape
    hdim = Ww1.shape[0]
    e = src.shape[0]
    c1_1 = Wv1[0, 0] * Wa1[0, 2 * hdim]
    c1_2 = Wv2[0, 0] * Wa2[0, 2 * hdim]

    z, zi, sd = _prep_call(n, hdim, indim)(x, Ww1, Wu1, _wsd(Wa1, hdim))
    outz, den = _edge_phase(z, sd[:, 0], sd[:, 1], src, dst, ea, c1_1, n, e)
    z, zi, sd = _fprep_call(n, hdim)(zi, outz, den, Ww2, Wu2, _wsd(Wa2, hdim))
    outz, den = _edge_phase(z, sd[:, 0], sd[:, 1], src, dst, ea, c1_2, n, e)
    return _fin_call(n, hdim)(zi, outz, den)
